# baseline (speedup 1.0000x reference)
import functools

import jax
import jax.numpy as jnp
from jax import lax
from jax.experimental import pallas as pl
from jax.experimental.pallas import tpu as pltpu
from jax.experimental.pallas import tpu_sc as plsc

_B, _HW, _C = 8, 56 * 56, 192
_NW = 32
_ROWS_PER_W = (_B * _HW) // _NW
_NCHUNK = 7
_CHUNK = _ROWS_PER_W // _NCHUNK
_W_PER_BATCH = _HW // _ROWS_PER_W
_LANES = 16

_mesh = plsc.VectorSubcoreMesh(core_axis_name="c", subcore_axis_name="s")


@functools.partial(
    pl.kernel,
    mesh=_mesh,
    out_type=jax.ShapeDtypeStruct((_B * _HW, _C), jnp.float32),
    scratch_types=[
        pltpu.VMEM((_NCHUNK, _CHUNK), jnp.int32),
        pltpu.VMEM((_CHUNK, _C), jnp.float32),
        pltpu.SemaphoreType.DMA,
    ],
    compiler_params=pltpu.CompilerParams(use_tc_tiling_on_sc=False),
)
def _shuffle_sc(x_hbm, r_hbm, out_hbm, idx_v, buf_v, sem):
    wid = lax.axis_index("s") * 2 + lax.axis_index("c")
    b = wid // _W_PER_BATCH
    off = (wid % _W_PER_BATCH) * _ROWS_PER_W

    for j in range(_NCHUNK):
        pltpu.sync_copy(r_hbm.at[pl.ds(off + j * _CHUNK, _CHUNK)], idx_v.at[j])

    base = b * _HW
    for j in range(_NCHUNK):
        for i in range(_CHUNK // _LANES):
            sl = pl.ds(i * _LANES, _LANES)
            idx_v[j, sl] = idx_v[j, sl] + base

    out_base = wid * _ROWS_PER_W
    for j in range(_NCHUNK):
        pltpu.async_copy(x_hbm.at[idx_v.at[j]], buf_v, sem).wait()
        pltpu.sync_copy(buf_v, out_hbm.at[pl.ds(out_base + j * _CHUNK, _CHUNK)])


def kernel(x, r):
    B, H, W, C = x.shape
    xf = x.reshape(B * H * W, C)
    out = _shuffle_sc(xf, r.astype(jnp.int32))
    return out.reshape(B, H, W, C)
